# Initial kernel scaffold; baseline (speedup 1.0000x reference)
import jax
import jax.numpy as jnp
from jax.experimental import pallas as pl


def kernel(x, edge_index, edge_weights, W1, b1, W2, b2, W3, b3):
    raise NotImplementedError("write your pallas kernel here")


# speedup vs baseline: 9.1700x; 9.1700x over previous
import functools

import jax
import jax.numpy as jnp
from jax import lax
from jax.experimental import pallas as pl
from jax.experimental.pallas import tpu as pltpu
from jax.experimental.pallas import tpu_sc as plsc

_N = 10000
_D = 128
_E = 320000
_NP = 10240
_ETOT = _E + _N
_CK = 128
_NT = 32
_CH = -(-_ETOT // (_NT * _CK))
_EP = _NT * _CH * _CK
_RPT = _NP // 16
_BR = 512

_mesh = lambda: plsc.VectorSubcoreMesh(core_axis_name="c", subcore_axis_name="s")


@functools.partial(
    pl.kernel,
    mesh=_mesh(),
    compiler_params=pltpu.CompilerParams(needs_layout_passes=False),
    out_type=jax.ShapeDtypeStruct((2, _NP), jnp.float32),
    scratch_types=[
        pltpu.VMEM((_CK,), jnp.int32),
        pltpu.VMEM((_CK,), jnp.float32),
        pltpu.VMEM((_RPT,), jnp.float32),
        pltpu.VMEM_SHARED((_NP,), jnp.float32),
    ],
)
def _deg_sc(d_hbm, w_hbm, z_hbm, out_hbm, didx_v, w_v, row_v, deg_sh):
    c = lax.axis_index("c")
    s = lax.axis_index("s")
    r0 = s * _RPT
    pltpu.sync_copy(z_hbm.at[pl.ds(r0, _RPT)], deg_sh.at[pl.ds(r0, _RPT)])
    plsc.subcore_barrier()
    base0 = (c * 16 + s) * (_CH * _CK)

    def chunk(j, carry):
        base = base0 + j * _CK
        pltpu.sync_copy(d_hbm.at[pl.ds(base, _CK)], didx_v)
        pltpu.sync_copy(w_hbm.at[pl.ds(base, _CK)], w_v)
        pltpu.sync_copy(w_v, deg_sh.at[didx_v], add=True)
        return carry

    lax.fori_loop(0, _CH, chunk, 0)
    plsc.subcore_barrier()
    pltpu.sync_copy(deg_sh.at[pl.ds(r0, _RPT)], row_v)
    pltpu.sync_copy(row_v, out_hbm.at[c, pl.ds(r0, _RPT)])


@functools.partial(
    pl.kernel,
    mesh=_mesh(),
    compiler_params=pltpu.CompilerParams(needs_layout_passes=False),
    out_type=jax.ShapeDtypeStruct((_EP,), jnp.float32),
    scratch_types=[
        pltpu.VMEM((_CK,), jnp.int32),
        pltpu.VMEM((_CK,), jnp.int32),
        pltpu.VMEM((_CK,), jnp.float32),
        pltpu.VMEM((_CK,), jnp.float32),
        pltpu.VMEM((_NP,), jnp.float32),
    ],
)
def _norm_sc(s_hbm, d_hbm, w_hbm, dis_hbm, out_hbm,
             sidx_v, didx_v, w_v, nrm_v, dis_v):
    c = lax.axis_index("c")
    s = lax.axis_index("s")
    pltpu.sync_copy(dis_hbm, dis_v)
    base0 = (c * 16 + s) * (_CH * _CK)

    def chunk(j, carry):
        base = base0 + j * _CK
        pltpu.sync_copy(s_hbm.at[pl.ds(base, _CK)], sidx_v)
        pltpu.sync_copy(d_hbm.at[pl.ds(base, _CK)], didx_v)
        pltpu.sync_copy(w_hbm.at[pl.ds(base, _CK)], w_v)
        for g in range(8):
            sl = pl.ds(g * 16, 16)
            ds_s = plsc.load_gather(dis_v, [sidx_v[sl]])
            ds_d = plsc.load_gather(dis_v, [didx_v[sl]])
            nrm_v[sl] = ds_s * w_v[sl] * ds_d
        pltpu.sync_copy(nrm_v, out_hbm.at[pl.ds(base, _CK)])
        return carry

    lax.fori_loop(0, _CH, chunk, 0)


@functools.partial(
    pl.kernel,
    mesh=_mesh(),
    compiler_params=pltpu.CompilerParams(needs_layout_passes=False),
    out_type=jax.ShapeDtypeStruct((2, _NP, _D), jnp.float32),
    scratch_types=[
        pltpu.VMEM((_CK,), jnp.int32),
        pltpu.VMEM((_CK,), jnp.int32),
        pltpu.VMEM((_CK,), jnp.float32),
        pltpu.VMEM((_CK, _D), jnp.float32),
        pltpu.VMEM_SHARED((_NP, _D), jnp.float32),
        pltpu.SemaphoreType.DMA,
    ],
)
def _agg_sc(xw_hbm, s_hbm, d_hbm, nrm_hbm, out_hbm,
            sidx_v, didx_v, nrm_v, rows_v, acc_sh, sem):
    c = lax.axis_index("c")
    s = lax.axis_index("s")

    def zrow(i, carry):
        z = jnp.zeros((16,), jnp.float32)
        for g in range(8):
            rows_v[i, pl.ds(g * 16, 16)] = z
        return carry

    lax.fori_loop(0, _CK, zrow, 0)
    for k in range(_RPT // _CK):
        pltpu.sync_copy(rows_v, acc_sh.at[pl.ds(s * _RPT + k * _CK, _CK)])
    plsc.subcore_barrier()

    base0 = (c * 16 + s) * (_CH * _CK)

    def chunk(j, carry):
        base = base0 + j * _CK
        pltpu.sync_copy(s_hbm.at[pl.ds(base, _CK)], sidx_v)
        pltpu.sync_copy(d_hbm.at[pl.ds(base, _CK)], didx_v)
        pltpu.sync_copy(nrm_hbm.at[pl.ds(base, _CK)], nrm_v)
        pltpu.async_copy(xw_hbm.at[sidx_v], rows_v, sem).wait()

        def scale(gq, carry2):
            nvv = nrm_v[pl.ds(gq * 16, 16)]
            for l in range(16):
                e = gq * 16 + l
                nv = nvv[l]
                for g in range(8):
                    sl = pl.ds(g * 16, 16)
                    rows_v[e, sl] = rows_v[e, sl] * nv
            return carry2

        lax.fori_loop(0, _CK // 16, scale, 0)
        pltpu.sync_copy(rows_v, acc_sh.at[didx_v], add=True)
        return carry

    lax.fori_loop(0, _CH, chunk, 0)
    plsc.subcore_barrier()
    for k in range(_RPT // _CK):
        r0 = s * _RPT + k * _CK
        pltpu.sync_copy(acc_sh.at[pl.ds(r0, _CK)], rows_v)
        pltpu.sync_copy(rows_v, out_hbm.at[c, pl.ds(r0, _CK)])


def _dis_body(degp_ref, o_ref):
    deg = degp_ref[0] + degp_ref[1]
    o_ref[...] = jnp.where(deg > 0, lax.rsqrt(deg), 0.0)


def _dis_tc(degp):
    return pl.pallas_call(
        _dis_body,
        out_shape=jax.ShapeDtypeStruct((_NP // _D, _D), jnp.float32),
    )(degp)


def _mm1_body(x_ref, w_ref, o_ref):
    o_ref[...] = jnp.dot(x_ref[...], w_ref[...],
                         preferred_element_type=jnp.float32)


def _mm1_tc(x, W):
    return pl.pallas_call(
        _mm1_body,
        grid=(_NP // _BR,),
        in_specs=[pl.BlockSpec((_BR, _D), lambda i: (i, 0)),
                  pl.BlockSpec((_D, _D), lambda i: (0, 0))],
        out_specs=pl.BlockSpec((_BR, _D), lambda i: (i, 0)),
        out_shape=jax.ShapeDtypeStruct((_NP, _D), jnp.float32),
    )(x, W)


def _mm2_body(p_ref, b_ref, w_ref, o_ref):
    h = jnp.maximum(p_ref[0] + p_ref[1] + b_ref[...], 0.0)
    o_ref[...] = jnp.dot(h, w_ref[...], preferred_element_type=jnp.float32)


def _mm2_tc(p, b, W):
    return pl.pallas_call(
        _mm2_body,
        grid=(_NP // _BR,),
        in_specs=[pl.BlockSpec((2, _BR, _D), lambda i: (0, i, 0)),
                  pl.BlockSpec((1, _D), lambda i: (0, 0)),
                  pl.BlockSpec((_D, _D), lambda i: (0, 0))],
        out_specs=pl.BlockSpec((_BR, _D), lambda i: (i, 0)),
        out_shape=jax.ShapeDtypeStruct((_NP, _D), jnp.float32),
    )(p, b, W)


def _mm3_body(p_ref, b2_ref, w_ref, b3_ref, o_ref):
    h = p_ref[0] + p_ref[1] + b2_ref[...]
    o_ref[...] = (jnp.dot(h, w_ref[...], preferred_element_type=jnp.float32)
                  + b3_ref[...])


def _mm3_tc(p, b2, W, b3):
    return pl.pallas_call(
        _mm3_body,
        grid=(_NP // _BR,),
        in_specs=[pl.BlockSpec((2, _BR, _D), lambda i: (0, i, 0)),
                  pl.BlockSpec((1, _D), lambda i: (0, 0)),
                  pl.BlockSpec((_D, _D), lambda i: (0, 0)),
                  pl.BlockSpec((1, _D), lambda i: (0, 0))],
        out_specs=pl.BlockSpec((_BR, _D), lambda i: (i, 0)),
        out_shape=jax.ShapeDtypeStruct((_NP, _D), jnp.float32),
    )(p, b2, W, b3)


def kernel(x, edge_index, edge_weights, W1, b1, W2, b2, W3, b3):
    src = edge_index[0].astype(jnp.int32)
    dst = edge_index[1].astype(jnp.int32)
    loop = jnp.arange(_N, dtype=jnp.int32)
    padi = jnp.zeros((_EP - _ETOT,), jnp.int32)
    padf = jnp.zeros((_EP - _ETOT,), jnp.float32)
    s_all = jnp.concatenate([src, loop, padi])
    d_all = jnp.concatenate([dst, loop, padi])
    w_all = jnp.concatenate([edge_weights, jnp.ones((_N,), jnp.float32), padf])
    x_pad = jnp.zeros((_NP, _D), jnp.float32).at[:_N].set(x)

    degp = _deg_sc(d_all, w_all, jnp.zeros((_NP,), jnp.float32))
    deg2 = degp.reshape(2, _NP // _D, _D)
    dis = _dis_tc(deg2).reshape(_NP)
    nrm = _norm_sc(s_all, d_all, w_all, dis)

    xw1 = _mm1_tc(x_pad, W1)
    p1 = _agg_sc(xw1, s_all, d_all, nrm)
    xw2 = _mm2_tc(p1, b1.reshape(1, _D), W2)
    p2 = _agg_sc(xw2, s_all, d_all, nrm)
    out = _mm3_tc(p2, b2.reshape(1, _D), W3, b3.reshape(1, _D))
    return out[:_N]
